# Initial kernel scaffold; baseline (speedup 1.0000x reference)
import functools

import jax
import jax.numpy as jnp
from jax import lax
from jax.experimental import pallas as pl
from jax.experimental.pallas import tpu as pltpu
from jax.experimental.pallas import tpu_sc as plsc

N = 10000
E = 320000
D = 128
H = 128
C = 16

NC = 2
NS = 16
NW = NC * NS
CH = 128
NCHUNK = 80
E_PAD = NW * NCHUNK * CH
N_PAD = 10240
RPT = N_PAD // NS

_f32 = jnp.float32


def _make_edge_scatter(feat, gather):
    rows_shape = (CH,) + feat
    feat_elems = 1
    for f in feat:
        feat_elems *= f
    nvec = (CH * feat_elems) // 16

    mesh = plsc.VectorSubcoreMesh(core_axis_name="c", subcore_axis_name="s")

    def body(*refs):
        if gather:
            hs_hbm, src_hbm, dst_hbm, out_hbm, src_v, dst_v, rows_v, acc_sh = refs
        else:
            dst_hbm, out_hbm, dst_v, rows_v, acc_sh = refs
        c = lax.axis_index("c")
        s = lax.axis_index("s")
        wid = s * NC + c

        def fill(val):
            def fbody(t, _):
                if feat == ():
                    rows_v[pl.ds(t * 16, 16)] = jnp.full((16,), val, _f32)
                else:
                    i = t // (feat_elems // 16)
                    k = t % (feat_elems // 16)
                    rows_v[i, pl.ds(k * 16, 16)] = jnp.full((16,), val, _f32)
                return 0

            lax.fori_loop(0, nvec, fbody, 0)

        fill(0.0)
        r0 = s * RPT
        for k in range(RPT // CH):
            pltpu.sync_copy(rows_v, acc_sh.at[pl.ds(r0 + k * CH, CH)])
        plsc.subcore_barrier()

        if gather:
            pltpu.sync_copy(src_hbm.at[wid], src_v)
        pltpu.sync_copy(dst_hbm.at[wid], dst_v)
        if not gather:
            fill(1.0)

        def chunk(j, _):
            if gather:
                pltpu.sync_copy(hs_hbm.at[src_v.at[j]], rows_v)
            pltpu.sync_copy(rows_v, acc_sh.at[dst_v.at[j]], add=True)
            return 0

        lax.fori_loop(0, NCHUNK, chunk, 0)
        plsc.subcore_barrier()

        for k in range(RPT // CH):
            pltpu.sync_copy(acc_sh.at[pl.ds(r0 + k * CH, CH)], rows_v)
            pltpu.sync_copy(rows_v, out_hbm.at[c, pl.ds(r0 + k * CH, CH)])

    scratch = []
    if gather:
        scratch.append(pltpu.VMEM((NCHUNK, CH), jnp.int32))
    scratch += [
        pltpu.VMEM((NCHUNK, CH), jnp.int32),
        pltpu.VMEM(rows_shape, _f32),
        pltpu.VMEM_SHARED((N_PAD,) + feat, _f32),
    ]

    kern = pl.kernel(
        body,
        out_type=jax.ShapeDtypeStruct((2, N_PAD) + feat, _f32),
        mesh=mesh,
        scratch_types=scratch,
    )
    return kern


_deg_scatter = _make_edge_scatter((), gather=False)
_scatter_h = _make_edge_scatter((H,), gather=True)
_scatter_c = _make_edge_scatter((C,), gather=True)


def _tc_first(x, w, degp):
    def body(x_ref, w_ref, deg_ref, hs_ref, dis_ref):
        dis = lax.rsqrt(deg_ref[0, :N, :] + deg_ref[1, :N, :] + 1.0)
        dis_ref[...] = dis
        hs_ref[...] = dis * jnp.dot(x_ref[...], w_ref[...],
                                    preferred_element_type=_f32)

    return pl.pallas_call(
        body,
        out_shape=(
            jax.ShapeDtypeStruct((N, w.shape[1]), _f32),
            jax.ShapeDtypeStruct((N, 1), _f32),
        ),
    )(x, w, degp)


def _tc_mid(p, hs, dis, b, w):
    def body(p_ref, hs_ref, dis_ref, b_ref, w_ref, out_ref):
        dis = dis_ref[...]
        a = dis * (p_ref[0, :N, :] + p_ref[1, :N, :] + hs_ref[...]) + b_ref[...]
        h = jnp.maximum(a, 0.0)
        out_ref[...] = dis * jnp.dot(h, w_ref[...], preferred_element_type=_f32)

    return pl.pallas_call(
        body,
        out_shape=jax.ShapeDtypeStruct((N, w.shape[1]), _f32),
    )(p, hs, dis, b, w)


def _tc_last(p, hs, dis, b):
    def body(p_ref, hs_ref, dis_ref, b_ref, out_ref):
        a = dis_ref[...] * (p_ref[0, :N, :] + p_ref[1, :N, :] + hs_ref[...]) \
            + b_ref[...]
        m = jnp.max(a, axis=1, keepdims=True)
        lse = m + jnp.log(jnp.sum(jnp.exp(a - m), axis=1, keepdims=True))
        out_ref[...] = a - lse

    return pl.pallas_call(
        body,
        out_shape=jax.ShapeDtypeStruct((N, C), _f32),
    )(p, hs, dis, b)


@jax.jit
def kernel(x, edge_index, W1, b1, W2, b2, W3, b3, W4, b4):
    pad = E_PAD - E
    src = jnp.concatenate(
        [edge_index[0], jnp.zeros((pad,), jnp.int32)]).reshape(NW, NCHUNK, CH)
    dst = jnp.concatenate(
        [edge_index[1], jnp.full((pad,), N, jnp.int32)]).reshape(NW, NCHUNK, CH)

    degp = _deg_scatter(dst).reshape(2, N_PAD, 1)
    hs1, dis = _tc_first(x, W1, degp)

    p = _scatter_h(hs1, src, dst)
    hs2 = _tc_mid(p, hs1, dis, b1.reshape(1, H), W2)
    p = _scatter_h(hs2, src, dst)
    hs3 = _tc_mid(p, hs2, dis, b2.reshape(1, H), W2)
    p = _scatter_h(hs3, src, dst)
    hs4 = _tc_mid(p, hs3, dis, b2.reshape(1, H), W3)
    p = _scatter_h(hs4, src, dst)
    hs5 = _tc_mid(p, hs4, dis, b3.reshape(1, H), W4)
    p = _scatter_c(hs5, src, dst)
    return _tc_last(p, hs5, dis, b4.reshape(1, C))


# speedup vs baseline: 5.2207x; 5.2207x over previous
import functools

import jax
import jax.numpy as jnp
from jax import lax
from jax.experimental import pallas as pl
from jax.experimental.pallas import tpu as pltpu
from jax.experimental.pallas import tpu_sc as plsc

N = 10000
E = 320000
D = 128
H = 128
C = 16

NC = 2
NS = 16
NW = NC * NS
CH = 128
NCHUNK = 80
E_PAD = NW * NCHUNK * CH
N_PAD = 10240
RPT = N_PAD // NS

_f32 = jnp.float32


def _make_edge_scatter(feat, gather):
    rows_shape = (CH,) + feat
    feat_elems = 1
    for f in feat:
        feat_elems *= f
    nvec = (CH * feat_elems) // 16

    mesh = plsc.VectorSubcoreMesh(core_axis_name="c", subcore_axis_name="s")

    def body(*refs):
        if gather:
            hs_hbm, src_hbm, dst_hbm, out_hbm, src_v, dst_v, rows_v, acc_sh = refs
        else:
            dst_hbm, out_hbm, dst_v, rows_v, acc_sh = refs
        c = lax.axis_index("c")
        s = lax.axis_index("s")
        wid = s * NC + c

        def fill(val):
            def fbody(t, _):
                if feat == ():
                    rows_v[pl.ds(t * 16, 16)] = jnp.full((16,), val, _f32)
                else:
                    i = t // (feat_elems // 16)
                    k = t % (feat_elems // 16)
                    rows_v[i, pl.ds(k * 16, 16)] = jnp.full((16,), val, _f32)
                return 0

            lax.fori_loop(0, nvec, fbody, 0)

        fill(0.0)
        r0 = s * RPT
        for k in range(RPT // CH):
            pltpu.sync_copy(rows_v, acc_sh.at[pl.ds(r0 + k * CH, CH)])
        plsc.subcore_barrier()

        if gather:
            pltpu.sync_copy(src_hbm.at[wid], src_v)
        pltpu.sync_copy(dst_hbm.at[wid], dst_v)
        if not gather:
            fill(1.0)

        def chunk(j, _):
            if gather:
                pltpu.sync_copy(hs_hbm.at[src_v.at[j]], rows_v)
            pltpu.sync_copy(rows_v, acc_sh.at[dst_v.at[j]], add=True)
            return 0

        lax.fori_loop(0, NCHUNK, chunk, 0)
        plsc.subcore_barrier()

        for k in range(RPT // CH):
            pltpu.sync_copy(acc_sh.at[pl.ds(r0 + k * CH, CH)], rows_v)
            pltpu.sync_copy(rows_v, out_hbm.at[c, pl.ds(r0 + k * CH, CH)])

    scratch = []
    if gather:
        scratch.append(pltpu.VMEM((NCHUNK, CH), jnp.int32))
    scratch += [
        pltpu.VMEM((NCHUNK, CH), jnp.int32),
        pltpu.VMEM(rows_shape, _f32),
        pltpu.VMEM_SHARED((N_PAD,) + feat, _f32),
    ]

    kern = pl.kernel(
        body,
        out_type=jax.ShapeDtypeStruct((2, N_PAD) + feat, _f32),
        mesh=mesh,
        scratch_types=scratch,
    )
    return kern


_deg_scatter = _make_edge_scatter((), gather=False)
_scatter_h = _make_edge_scatter((H,), gather=True)


def _tc_first(x, w, degp):
    def body(x_ref, w_ref, deg_ref, hs_ref, dis_ref):
        dis = lax.rsqrt(deg_ref[0, :N, :] + deg_ref[1, :N, :] + 1.0)
        dis_ref[...] = dis
        hs_ref[...] = dis * jnp.dot(x_ref[...], w_ref[...],
                                    preferred_element_type=_f32)

    return pl.pallas_call(
        body,
        out_shape=(
            jax.ShapeDtypeStruct((N, w.shape[1]), _f32),
            jax.ShapeDtypeStruct((N, 1), _f32),
        ),
    )(x, w, degp)


def _tc_mid(p, hs, dis, b, w):
    def body(p_ref, hs_ref, dis_ref, b_ref, w_ref, out_ref):
        dis = dis_ref[...]
        a = dis * (p_ref[0, :N, :] + p_ref[1, :N, :] + hs_ref[...]) + b_ref[...]
        h = jnp.maximum(a, 0.0)
        out_ref[...] = dis * jnp.dot(h, w_ref[...], preferred_element_type=_f32)

    return pl.pallas_call(
        body,
        out_shape=jax.ShapeDtypeStruct((N, w.shape[1]), _f32),
    )(p, hs, dis, b, w)


def _tc_last(p, hs, dis, b):
    def body(p_ref, hs_ref, dis_ref, b_ref, out_ref):
        a = dis_ref[...] * (p_ref[0, :N, :C] + p_ref[1, :N, :C]
                            + hs_ref[:, :C]) + b_ref[...]
        m = jnp.max(a, axis=1, keepdims=True)
        lse = m + jnp.log(jnp.sum(jnp.exp(a - m), axis=1, keepdims=True))
        out_ref[...] = a - lse

    return pl.pallas_call(
        body,
        out_shape=jax.ShapeDtypeStruct((N, C), _f32),
    )(p, hs, dis, b)


@jax.jit
def kernel(x, edge_index, W1, b1, W2, b2, W3, b3, W4, b4):
    pad = E_PAD - E
    src = jnp.concatenate(
        [edge_index[0], jnp.zeros((pad,), jnp.int32)]).reshape(NW, NCHUNK, CH)
    dst = jnp.concatenate(
        [edge_index[1], jnp.full((pad,), N, jnp.int32)]).reshape(NW, NCHUNK, CH)

    degp = _deg_scatter(dst).reshape(2, N_PAD, 1)
    hs1, dis = _tc_first(x, W1, degp)

    p = _scatter_h(hs1, src, dst)
    hs2 = _tc_mid(p, hs1, dis, b1.reshape(1, H), W2)
    p = _scatter_h(hs2, src, dst)
    hs3 = _tc_mid(p, hs2, dis, b2.reshape(1, H), W2)
    p = _scatter_h(hs3, src, dst)
    hs4 = _tc_mid(p, hs3, dis, b2.reshape(1, H), W3)
    p = _scatter_h(hs4, src, dst)
    W4p = jnp.pad(W4, ((0, 0), (0, H - C)))
    hs5 = _tc_mid(p, hs4, dis, b3.reshape(1, H), W4p)
    p = _scatter_h(hs5, src, dst)
    return _tc_last(p, hs5, dis, b4.reshape(1, C))


# speedup vs baseline: 5.6815x; 1.0883x over previous
import functools

import jax
import jax.numpy as jnp
from jax import lax
from jax.experimental import pallas as pl
from jax.experimental.pallas import tpu as pltpu
from jax.experimental.pallas import tpu_sc as plsc

N = 10000
E = 320000
D = 128
H = 128
C = 16

NC = 2
NS = 16
NW = NC * NS
CH = 128
NCHUNK = 80
MBLK = 40
E_PAD = NW * NCHUNK * CH
N_PAD = 10240
RPT = N_PAD // NS

_f32 = jnp.float32


def _make_edge_scatter(feat, gather):
    rows_shape = (CH,) + feat
    feat_elems = 1
    for f in feat:
        feat_elems *= f
    nvec = (CH * feat_elems) // 16

    mesh = plsc.VectorSubcoreMesh(core_axis_name="c", subcore_axis_name="s")

    def body(*refs):
        if gather:
            (hs_hbm, src_hbm, dst_hbm, out_hbm, src_v, dst_v, rows_v, acc_sh,
             gsem0, gsem1, ssem0, ssem1) = refs
            rv0 = rows_v.at[0]
        else:
            dst_hbm, out_hbm, dst_v, rows_v, acc_sh = refs
            rv0 = rows_v
        c = lax.axis_index("c")
        s = lax.axis_index("s")
        wid = s * NC + c

        def fill(val):
            def fbody(t, _):
                if feat == ():
                    rows_v[pl.ds(t * 16, 16)] = jnp.full((16,), val, _f32)
                else:
                    i = t // (feat_elems // 16)
                    k = t % (feat_elems // 16)
                    rows_v[0, i, pl.ds(k * 16, 16)] = jnp.full((16,), val, _f32)
                return 0

            lax.fori_loop(0, nvec, fbody, 0)

        fill(0.0)
        r0 = s * RPT
        for k in range(RPT // CH):
            pltpu.sync_copy(rv0, acc_sh.at[pl.ds(r0 + k * CH, CH)])
        plsc.subcore_barrier()

        if gather:
            gsems = [gsem0, gsem1]
            ssems = [ssem0, ssem1]

            def g_start(j, buf):
                pltpu.async_copy(hs_hbm.at[src_v.at[j]], rows_v.at[buf],
                                 gsems[buf])

            def g_wait(j, buf):
                pltpu.make_async_copy(hs_hbm.at[src_v.at[j]],
                                      rows_v.at[buf], gsems[buf]).wait()

            def s_start(j, buf):
                pltpu.async_copy(rows_v.at[buf], acc_sh.at[dst_v.at[j]],
                                 ssems[buf], add=True)

            def s_wait(j, buf):
                pltpu.make_async_copy(rows_v.at[buf], acc_sh.at[dst_v.at[j]],
                                      ssems[buf]).wait()

            for b in range(NCHUNK // MBLK):
                pltpu.sync_copy(src_hbm.at[wid, pl.ds(b * MBLK, MBLK)], src_v)
                pltpu.sync_copy(dst_hbm.at[wid, pl.ds(b * MBLK, MBLK)], dst_v)
                g_start(0, 0)
                g_start(1, 1)

                def chunk(it, _):
                    j = 2 * it
                    for buf in range(2):
                        g_wait(j + buf, buf)
                        s_start(j + buf, buf)
                    for buf in range(2):
                        s_wait(j + buf, buf)

                        @pl.when(j + buf + 2 < MBLK)
                        def _():
                            g_start(j + buf + 2, buf)

                    return 0

                lax.fori_loop(0, MBLK // 2, chunk, 0)
        else:
            pltpu.sync_copy(dst_hbm.at[wid], dst_v)
            fill(1.0)

            def chunk(j, _):
                pltpu.sync_copy(rows_v, acc_sh.at[dst_v.at[j]], add=True)
                return 0

            lax.fori_loop(0, NCHUNK, chunk, 0)
        plsc.subcore_barrier()

        for k in range(RPT // CH):
            pltpu.sync_copy(acc_sh.at[pl.ds(r0 + k * CH, CH)], rv0)
            pltpu.sync_copy(rv0, out_hbm.at[c, pl.ds(r0 + k * CH, CH)])

    scratch = []
    if gather:
        scratch.append(pltpu.VMEM((MBLK, CH), jnp.int32))
    scratch += [
        pltpu.VMEM((MBLK if gather else NCHUNK, CH), jnp.int32),
        pltpu.VMEM(((2,) if gather else ()) + rows_shape, _f32),
        pltpu.VMEM_SHARED((N_PAD,) + feat, _f32),
    ]
    if gather:
        scratch += [pltpu.SemaphoreType.DMA] * 4

    kern = pl.kernel(
        body,
        out_type=jax.ShapeDtypeStruct((2, N_PAD) + feat, _f32),
        mesh=mesh,
        scratch_types=scratch,
    )
    return kern


_deg_scatter = _make_edge_scatter((), gather=False)
_scatter_h = _make_edge_scatter((H,), gather=True)


def _tc_first(x, w, degp):
    def body(x_ref, w_ref, deg_ref, hs_ref, dis_ref):
        dis = lax.rsqrt(deg_ref[0, :N, :] + deg_ref[1, :N, :] + 1.0)
        dis_ref[...] = dis
        hs_ref[...] = dis * jnp.dot(x_ref[...], w_ref[...],
                                    preferred_element_type=_f32)

    return pl.pallas_call(
        body,
        out_shape=(
            jax.ShapeDtypeStruct((N, w.shape[1]), _f32),
            jax.ShapeDtypeStruct((N, 1), _f32),
        ),
    )(x, w, degp)


def _tc_mid(p, hs, dis, b, w):
    def body(p_ref, hs_ref, dis_ref, b_ref, w_ref, out_ref):
        dis = dis_ref[...]
        a = dis * (p_ref[0, :N, :] + p_ref[1, :N, :] + hs_ref[...]) + b_ref[...]
        h = jnp.maximum(a, 0.0)
        out_ref[...] = dis * jnp.dot(h, w_ref[...], preferred_element_type=_f32)

    return pl.pallas_call(
        body,
        out_shape=jax.ShapeDtypeStruct((N, w.shape[1]), _f32),
    )(p, hs, dis, b, w)


def _tc_last(p, hs, dis, b):
    def body(p_ref, hs_ref, dis_ref, b_ref, out_ref):
        a = dis_ref[...] * (p_ref[0, :N, :C] + p_ref[1, :N, :C]
                            + hs_ref[:, :C]) + b_ref[...]
        m = jnp.max(a, axis=1, keepdims=True)
        lse = m + jnp.log(jnp.sum(jnp.exp(a - m), axis=1, keepdims=True))
        out_ref[...] = a - lse

    return pl.pallas_call(
        body,
        out_shape=jax.ShapeDtypeStruct((N, C), _f32),
    )(p, hs, dis, b)


@jax.jit
def kernel(x, edge_index, W1, b1, W2, b2, W3, b3, W4, b4):
    pad = E_PAD - E
    src = jnp.concatenate(
        [edge_index[0], jnp.zeros((pad,), jnp.int32)]).reshape(NW, NCHUNK, CH)
    dst = jnp.concatenate(
        [edge_index[1], jnp.full((pad,), N, jnp.int32)]).reshape(NW, NCHUNK, CH)

    degp = _deg_scatter(dst).reshape(2, N_PAD, 1)
    hs1, dis = _tc_first(x, W1, degp)

    p = _scatter_h(hs1, src, dst)
    hs2 = _tc_mid(p, hs1, dis, b1.reshape(1, H), W2)
    p = _scatter_h(hs2, src, dst)
    hs3 = _tc_mid(p, hs2, dis, b2.reshape(1, H), W2)
    p = _scatter_h(hs3, src, dst)
    hs4 = _tc_mid(p, hs3, dis, b2.reshape(1, H), W3)
    p = _scatter_h(hs4, src, dst)
    W4p = jnp.pad(W4, ((0, 0), (0, H - C)))
    hs5 = _tc_mid(p, hs4, dis, b3.reshape(1, H), W4p)
    p = _scatter_h(hs5, src, dst)
    return _tc_last(p, hs5, dis, b4.reshape(1, C))


# speedup vs baseline: 7.9221x; 1.3944x over previous
import functools

import jax
import jax.numpy as jnp
from jax import lax
from jax.experimental import pallas as pl
from jax.experimental.pallas import tpu as pltpu
from jax.experimental.pallas import tpu_sc as plsc

N = 10000
E = 320000
D = 128
H = 128
C = 16

NC = 2
NS = 16
NW = NC * NS
CH = 128
NCHUNK = 80
MBLK = 40
NCHUNK_TOT = 2560
E_PAD = NCHUNK_TOT * CH
N_PAD = 10240
RPT = N_PAD // NS
Q_FAST = 120
Q_SLOW = 40
FAST_C = 0

_f32 = jnp.float32


def _make_edge_scatter(feat, gather):
    rows_shape = (CH,) + feat
    feat_elems = 1
    for f in feat:
        feat_elems *= f
    nvec = (CH * feat_elems) // 16

    mesh = plsc.VectorSubcoreMesh(core_axis_name="c", subcore_axis_name="s")

    def body(*refs):
        if gather:
            (hs_hbm, src_hbm, dst_hbm, out_hbm, src_v, dst_v, rows_v, acc_sh,
             gsem0, gsem1, ssem0, ssem1) = refs
            rv0 = rows_v.at[0]
        else:
            dst_hbm, out_hbm, dst_v, rows_v, acc_sh = refs
            rv0 = rows_v
        c = lax.axis_index("c")
        s = lax.axis_index("s")
        wid = s * NC + c

        def fill(val):
            def fbody(t, _):
                if feat == ():
                    rows_v[pl.ds(t * 16, 16)] = jnp.full((16,), val, _f32)
                else:
                    i = t // (feat_elems // 16)
                    k = t % (feat_elems // 16)
                    rows_v[0, i, pl.ds(k * 16, 16)] = jnp.full((16,), val, _f32)
                return 0

            lax.fori_loop(0, nvec, fbody, 0)

        fill(0.0)
        r0 = s * RPT
        for k in range(RPT // CH):
            pltpu.sync_copy(rv0, acc_sh.at[pl.ds(r0 + k * CH, CH)])
        plsc.subcore_barrier()

        if gather:
            gsems = [gsem0, gsem1]
            ssems = [ssem0, ssem1]

            def g_start(j, buf):
                pltpu.async_copy(hs_hbm.at[src_v.at[j]], rows_v.at[buf],
                                 gsems[buf])

            def g_wait(j, buf):
                pltpu.make_async_copy(hs_hbm.at[src_v.at[j]],
                                      rows_v.at[buf], gsems[buf]).wait()

            def s_start(j, buf):
                pltpu.async_copy(rows_v.at[buf], acc_sh.at[dst_v.at[j]],
                                 ssems[buf], add=True)

            def s_wait(j, buf):
                pltpu.make_async_copy(rows_v.at[buf], acc_sh.at[dst_v.at[j]],
                                      ssems[buf]).wait()

            is_fast = (c == FAST_C)
            start0 = jnp.where(is_fast, s * Q_FAST, NS * Q_FAST + s * Q_SLOW)
            nblk = jnp.where(is_fast, Q_FAST // MBLK, Q_SLOW // MBLK)

            def block(b, _):
                cs = start0 + b * MBLK
                pltpu.sync_copy(src_hbm.at[pl.ds(cs, MBLK)], src_v)
                pltpu.sync_copy(dst_hbm.at[pl.ds(cs, MBLK)], dst_v)
                g_start(0, 0)
                g_start(1, 1)

                def chunk(it, _):
                    j = 2 * it
                    for buf in range(2):
                        g_wait(j + buf, buf)
                        s_start(j + buf, buf)
                    for buf in range(2):
                        s_wait(j + buf, buf)

                        @pl.when(j + buf + 2 < MBLK)
                        def _():
                            g_start(j + buf + 2, buf)

                    return 0

                lax.fori_loop(0, MBLK // 2, chunk, 0)
                return 0

            lax.fori_loop(0, nblk, block, 0)
        else:
            pltpu.sync_copy(dst_hbm.at[pl.ds(wid * NCHUNK, NCHUNK)], dst_v)
            fill(1.0)

            def chunk(j, _):
                pltpu.sync_copy(rows_v, acc_sh.at[dst_v.at[j]], add=True)
                return 0

            lax.fori_loop(0, NCHUNK, chunk, 0)
        plsc.subcore_barrier()

        for k in range(RPT // CH):
            pltpu.sync_copy(acc_sh.at[pl.ds(r0 + k * CH, CH)], rv0)
            pltpu.sync_copy(rv0, out_hbm.at[c, pl.ds(r0 + k * CH, CH)])

    scratch = []
    if gather:
        scratch.append(pltpu.VMEM((MBLK, CH), jnp.int32))
    scratch += [
        pltpu.VMEM((MBLK if gather else NCHUNK, CH), jnp.int32),
        pltpu.VMEM(((2,) if gather else ()) + rows_shape, _f32),
        pltpu.VMEM_SHARED((N_PAD,) + feat, _f32),
    ]
    if gather:
        scratch += [pltpu.SemaphoreType.DMA] * 4

    kern = pl.kernel(
        body,
        out_type=jax.ShapeDtypeStruct((2, N_PAD) + feat, _f32),
        mesh=mesh,
        scratch_types=scratch,
    )
    return kern


_deg_scatter = _make_edge_scatter((), gather=False)
_scatter_h = _make_edge_scatter((H,), gather=True)


def _tc_first(x, w, degp):
    def body(x_ref, w_ref, deg_ref, hs_ref, dis_ref):
        dis = lax.rsqrt(deg_ref[0, :N, :] + deg_ref[1, :N, :] + 1.0)
        dis_ref[...] = dis
        hs_ref[...] = dis * jnp.dot(x_ref[...], w_ref[...],
                                    preferred_element_type=_f32)

    return pl.pallas_call(
        body,
        out_shape=(
            jax.ShapeDtypeStruct((N, w.shape[1]), _f32),
            jax.ShapeDtypeStruct((N, 1), _f32),
        ),
    )(x, w, degp)


def _tc_mid(p, hs, dis, b, w):
    def body(p_ref, hs_ref, dis_ref, b_ref, w_ref, out_ref):
        dis = dis_ref[...]
        a = dis * (p_ref[0, :N, :] + p_ref[1, :N, :] + hs_ref[...]) + b_ref[...]
        h = jnp.maximum(a, 0.0)
        out_ref[...] = dis * jnp.dot(h, w_ref[...], preferred_element_type=_f32)

    return pl.pallas_call(
        body,
        out_shape=jax.ShapeDtypeStruct((N, w.shape[1]), _f32),
    )(p, hs, dis, b, w)


def _tc_last(p, hs, dis, b):
    def body(p_ref, hs_ref, dis_ref, b_ref, out_ref):
        a = dis_ref[...] * (p_ref[0, :N, :C] + p_ref[1, :N, :C]
                            + hs_ref[:, :C]) + b_ref[...]
        m = jnp.max(a, axis=1, keepdims=True)
        lse = m + jnp.log(jnp.sum(jnp.exp(a - m), axis=1, keepdims=True))
        out_ref[...] = a - lse

    return pl.pallas_call(
        body,
        out_shape=jax.ShapeDtypeStruct((N, C), _f32),
    )(p, hs, dis, b)


@jax.jit
def kernel(x, edge_index, W1, b1, W2, b2, W3, b3, W4, b4):
    pad = E_PAD - E
    src = jnp.concatenate(
        [edge_index[0], jnp.zeros((pad,), jnp.int32)]).reshape(NCHUNK_TOT, CH)
    dst = jnp.concatenate(
        [edge_index[1], jnp.full((pad,), N, jnp.int32)]).reshape(NCHUNK_TOT, CH)

    degp = _deg_scatter(dst).reshape(2, N_PAD, 1)
    hs1, dis = _tc_first(x, W1, degp)

    p = _scatter_h(hs1, src, dst)
    hs2 = _tc_mid(p, hs1, dis, b1.reshape(1, H), W2)
    p = _scatter_h(hs2, src, dst)
    hs3 = _tc_mid(p, hs2, dis, b2.reshape(1, H), W2)
    p = _scatter_h(hs3, src, dst)
    hs4 = _tc_mid(p, hs3, dis, b2.reshape(1, H), W3)
    p = _scatter_h(hs4, src, dst)
    W4p = jnp.pad(W4, ((0, 0), (0, H - C)))
    hs5 = _tc_mid(p, hs4, dis, b3.reshape(1, H), W4p)
    p = _scatter_h(hs5, src, dst)
    return _tc_last(p, hs5, dis, b4.reshape(1, C))


# speedup vs baseline: 8.1489x; 1.0286x over previous
import functools

import jax
import jax.numpy as jnp
from jax import lax
from jax.experimental import pallas as pl
from jax.experimental.pallas import tpu as pltpu
from jax.experimental.pallas import tpu_sc as plsc

N = 10000
E = 320000
D = 128
H = 128
C = 16

NC = 2
NS = 16
NW = NC * NS
CH = 128
NCHUNK = 80
MBLK = 16
NCHUNK_TOT = 2560
E_PAD = NCHUNK_TOT * CH
N_PAD = 10240
RPT = N_PAD // NS
Q_FAST = 144
Q_SLOW = 16
FAST_C = 0

_f32 = jnp.float32


def _make_edge_scatter(feat, gather):
    rows_shape = (CH,) + feat
    feat_elems = 1
    for f in feat:
        feat_elems *= f
    nvec = (CH * feat_elems) // 16

    mesh = plsc.VectorSubcoreMesh(core_axis_name="c", subcore_axis_name="s")

    def body(*refs):
        if gather:
            (hs_hbm, src_hbm, dst_hbm, out_hbm, src_v, dst_v, rows_v, acc_sh,
             gsem0, gsem1, ssem0, ssem1) = refs
            rv0 = rows_v.at[0]
        else:
            dst_hbm, out_hbm, dst_v, rows_v, acc_sh = refs
            rv0 = rows_v
        c = lax.axis_index("c")
        s = lax.axis_index("s")
        wid = s * NC + c

        def fill(val):
            def fbody(t, _):
                if feat == ():
                    rows_v[pl.ds(t * 16, 16)] = jnp.full((16,), val, _f32)
                else:
                    i = t // (feat_elems // 16)
                    k = t % (feat_elems // 16)
                    rows_v[0, i, pl.ds(k * 16, 16)] = jnp.full((16,), val, _f32)
                return 0

            lax.fori_loop(0, nvec, fbody, 0)

        fill(0.0)
        r0 = s * RPT
        for k in range(RPT // CH):
            pltpu.sync_copy(rv0, acc_sh.at[pl.ds(r0 + k * CH, CH)])
        plsc.subcore_barrier()

        if gather:
            gsems = [gsem0, gsem1]
            ssems = [ssem0, ssem1]

            def g_start(j, buf):
                pltpu.async_copy(hs_hbm.at[src_v.at[j]], rows_v.at[buf],
                                 gsems[buf])

            def g_wait(j, buf):
                pltpu.make_async_copy(hs_hbm.at[src_v.at[j]],
                                      rows_v.at[buf], gsems[buf]).wait()

            def s_start(j, buf):
                pltpu.async_copy(rows_v.at[buf], acc_sh.at[dst_v.at[j]],
                                 ssems[buf], add=True)

            def s_wait(j, buf):
                pltpu.make_async_copy(rows_v.at[buf], acc_sh.at[dst_v.at[j]],
                                      ssems[buf]).wait()

            is_fast = (c == FAST_C)
            start0 = jnp.where(is_fast, s * Q_FAST, NS * Q_FAST + s * Q_SLOW)
            nblk = jnp.where(is_fast, Q_FAST // MBLK, Q_SLOW // MBLK)

            def block(b, _):
                cs = start0 + b * MBLK
                pltpu.sync_copy(src_hbm.at[pl.ds(cs, MBLK)], src_v)
                pltpu.sync_copy(dst_hbm.at[pl.ds(cs, MBLK)], dst_v)
                g_start(0, 0)
                g_start(1, 1)

                def chunk(it, _):
                    j = 2 * it
                    for buf in range(2):
                        g_wait(j + buf, buf)
                        s_start(j + buf, buf)
                    for buf in range(2):
                        s_wait(j + buf, buf)

                        @pl.when(j + buf + 2 < MBLK)
                        def _():
                            g_start(j + buf + 2, buf)

                    return 0

                lax.fori_loop(0, MBLK // 2, chunk, 0)
                return 0

            lax.fori_loop(0, nblk, block, 0)
        else:
            pltpu.sync_copy(dst_hbm.at[pl.ds(wid * NCHUNK, NCHUNK)], dst_v)
            fill(1.0)

            def chunk(j, _):
                pltpu.sync_copy(rows_v, acc_sh.at[dst_v.at[j]], add=True)
                return 0

            lax.fori_loop(0, NCHUNK, chunk, 0)
        plsc.subcore_barrier()

        for k in range(RPT // CH):
            pltpu.sync_copy(acc_sh.at[pl.ds(r0 + k * CH, CH)], rv0)
            pltpu.sync_copy(rv0, out_hbm.at[c, pl.ds(r0 + k * CH, CH)])

    scratch = []
    if gather:
        scratch.append(pltpu.VMEM((MBLK, CH), jnp.int32))
    scratch += [
        pltpu.VMEM((MBLK if gather else NCHUNK, CH), jnp.int32),
        pltpu.VMEM(((2,) if gather else ()) + rows_shape, _f32),
        pltpu.VMEM_SHARED((N_PAD,) + feat, _f32),
    ]
    if gather:
        scratch += [pltpu.SemaphoreType.DMA] * 4

    kern = pl.kernel(
        body,
        out_type=jax.ShapeDtypeStruct((2, N_PAD) + feat, _f32),
        mesh=mesh,
        scratch_types=scratch,
    )
    return kern


_deg_scatter = _make_edge_scatter((), gather=False)
_scatter_h = _make_edge_scatter((H,), gather=True)


def _tc_first(x, w, degp):
    def body(x_ref, w_ref, deg_ref, hs_ref, dis_ref):
        dis = lax.rsqrt(deg_ref[0, :N, :] + deg_ref[1, :N, :] + 1.0)
        dis_ref[...] = dis
        hs_ref[...] = dis * jnp.dot(x_ref[...], w_ref[...],
                                    preferred_element_type=_f32)

    return pl.pallas_call(
        body,
        out_shape=(
            jax.ShapeDtypeStruct((N, w.shape[1]), _f32),
            jax.ShapeDtypeStruct((N, 1), _f32),
        ),
    )(x, w, degp)


def _tc_mid(p, hs, dis, b, w):
    def body(p_ref, hs_ref, dis_ref, b_ref, w_ref, out_ref):
        dis = dis_ref[...]
        a = dis * (p_ref[0, :N, :] + p_ref[1, :N, :] + hs_ref[...]) + b_ref[...]
        h = jnp.maximum(a, 0.0)
        out_ref[...] = dis * jnp.dot(h, w_ref[...], preferred_element_type=_f32)

    return pl.pallas_call(
        body,
        out_shape=jax.ShapeDtypeStruct((N, w.shape[1]), _f32),
    )(p, hs, dis, b, w)


def _tc_last(p, hs, dis, b):
    def body(p_ref, hs_ref, dis_ref, b_ref, out_ref):
        a = dis_ref[...] * (p_ref[0, :N, :C] + p_ref[1, :N, :C]
                            + hs_ref[:, :C]) + b_ref[...]
        m = jnp.max(a, axis=1, keepdims=True)
        lse = m + jnp.log(jnp.sum(jnp.exp(a - m), axis=1, keepdims=True))
        out_ref[...] = a - lse

    return pl.pallas_call(
        body,
        out_shape=jax.ShapeDtypeStruct((N, C), _f32),
    )(p, hs, dis, b)


@jax.jit
def kernel(x, edge_index, W1, b1, W2, b2, W3, b3, W4, b4):
    pad = E_PAD - E
    src = jnp.concatenate(
        [edge_index[0], jnp.zeros((pad,), jnp.int32)]).reshape(NCHUNK_TOT, CH)
    dst = jnp.concatenate(
        [edge_index[1], jnp.full((pad,), N, jnp.int32)]).reshape(NCHUNK_TOT, CH)

    degp = _deg_scatter(dst).reshape(2, N_PAD, 1)
    hs1, dis = _tc_first(x, W1, degp)

    p = _scatter_h(hs1, src, dst)
    hs2 = _tc_mid(p, hs1, dis, b1.reshape(1, H), W2)
    p = _scatter_h(hs2, src, dst)
    hs3 = _tc_mid(p, hs2, dis, b2.reshape(1, H), W2)
    p = _scatter_h(hs3, src, dst)
    hs4 = _tc_mid(p, hs3, dis, b2.reshape(1, H), W3)
    p = _scatter_h(hs4, src, dst)
    W4p = jnp.pad(W4, ((0, 0), (0, H - C)))
    hs5 = _tc_mid(p, hs4, dis, b3.reshape(1, H), W4p)
    p = _scatter_h(hs5, src, dst)
    return _tc_last(p, hs5, dis, b4.reshape(1, C))
